# baseline (device time: 7515 ns/iter reference)
import jax
import jax.numpy as jnp
from jax import lax
from jax.experimental import pallas as pl
from jax.experimental.pallas import tpu as pltpu

N_DEV = 4
GRID = 8


def kernel(x):
    m_per, n = x.shape
    bm = m_per // GRID

    def body(x_ref, out_ref, best_ref, comm_ref, send_sems, recv_sems):
        my_pos = lax.axis_index("i")
        c = pl.program_id(0)

        barrier_sem = pltpu.get_barrier_semaphore()

        @pl.when(c == 0)
        def _():
            for d in range(1, N_DEV):
                peer = lax.rem(my_pos + d, N_DEV)
                pl.semaphore_signal(
                    barrier_sem, inc=1,
                    device_id=(peer,), device_id_type=pl.DeviceIdType.MESH,
                )

        xv = x_ref[:, :]
        vmax = jnp.max(xv, axis=0)
        row_iota = lax.broadcasted_iota(jnp.int32, xv.shape, 0)
        masked = jnp.where(xv == vmax[None, :], row_iota, jnp.int32(bm))
        vidx = (
            (jnp.min(masked, axis=0) + c * bm).astype(jnp.float32)
            + my_pos.astype(jnp.float32) * m_per
        )

        @pl.when(c == 0)
        def _():
            best_ref[0, :] = vmax
            best_ref[1, :] = vidx

        @pl.when(c > 0)
        def _():
            bv = best_ref[0, :]
            take = vmax > bv
            best_ref[0, :] = jnp.where(take, vmax, bv)
            best_ref[1, :] = jnp.where(take, vidx, best_ref[1, :])

        @pl.when(c == GRID - 1)
        def _():
            val = best_ref[0, :]
            idx = best_ref[1, :]
            comm_ref[N_DEV - 1, 0, :] = val
            comm_ref[N_DEV - 1, 1, :] = idx

            pl.semaphore_wait(barrier_sem, N_DEV - 1)

            rdmas = []
            for d in range(1, N_DEV):
                peer = lax.rem(my_pos + d, N_DEV)
                rdma = pltpu.make_async_remote_copy(
                    src_ref=comm_ref.at[N_DEV - 1],
                    dst_ref=comm_ref.at[d - 1],
                    send_sem=send_sems.at[d - 1],
                    recv_sem=recv_sems.at[d - 1],
                    device_id=(peer,),
                    device_id_type=pl.DeviceIdType.MESH,
                )
                rdma.start()
                rdmas.append(rdma)

            best_val = val
            best_idx = idx
            for d in (1, 3, 2):
                rdmas[d - 1].wait_recv()
                new_val = comm_ref[d - 1, 0, :]
                new_idx = comm_ref[d - 1, 1, :]
                take = (new_val > best_val) | (
                    (new_val == best_val) & (new_idx < best_idx)
                )
                best_val = jnp.where(take, new_val, best_val)
                best_idx = jnp.where(take, new_idx, best_idx)

            out_ref[0, :] = best_val
            out_ref[1, :] = best_idx

            for d in range(1, N_DEV):
                rdmas[d - 1].wait_send()

    return pl.pallas_call(
        body,
        grid=(GRID,),
        out_shape=jax.ShapeDtypeStruct((2, n), jnp.float32),
        in_specs=[
            pl.BlockSpec((bm, n), lambda c: (c, 0), memory_space=pltpu.VMEM)
        ],
        out_specs=pl.BlockSpec((2, n), lambda c: (0, 0), memory_space=pltpu.VMEM),
        scratch_shapes=[
            pltpu.VMEM((2, n), jnp.float32),
            pltpu.VMEM((N_DEV, 2, n), jnp.float32),
            pltpu.SemaphoreType.DMA((N_DEV - 1,)),
            pltpu.SemaphoreType.DMA((N_DEV - 1,)),
        ],
        compiler_params=pltpu.CompilerParams(
            collective_id=0,
            dimension_semantics=("arbitrary",),
        ),
    )(x)


# device time: 7367 ns/iter; 1.0201x vs baseline; 1.0201x over previous
import jax
import jax.numpy as jnp
from jax import lax
from jax.experimental import pallas as pl
from jax.experimental.pallas import tpu as pltpu

N_DEV = 4
GRID = 4


def kernel(x):
    m_per, n = x.shape
    bm = m_per // GRID

    def body(x_ref, out_ref, best_ref, comm_ref, send_sems, recv_sems, out_sem):
        my_pos = lax.axis_index("i")
        c = pl.program_id(0)

        barrier_sem = pltpu.get_barrier_semaphore()

        @pl.when(c == 0)
        def _():
            for d in range(1, N_DEV):
                peer = lax.rem(my_pos + d, N_DEV)
                pl.semaphore_signal(
                    barrier_sem, inc=1,
                    device_id=(peer,), device_id_type=pl.DeviceIdType.MESH,
                )

        xv = x_ref[:, :]
        vmax = jnp.max(xv, axis=0)
        row_iota = lax.broadcasted_iota(jnp.int32, xv.shape, 0)
        masked = jnp.where(xv == vmax[None, :], row_iota, jnp.int32(bm))
        vidx = (
            (jnp.min(masked, axis=0) + c * bm).astype(jnp.float32)
            + my_pos.astype(jnp.float32) * m_per
        )

        @pl.when(c == 0)
        def _():
            best_ref[0, :] = vmax
            best_ref[1, :] = vidx

        @pl.when(c > 0)
        def _():
            bv = best_ref[0, :]
            take = vmax > bv
            best_ref[0, :] = jnp.where(take, vmax, bv)
            best_ref[1, :] = jnp.where(take, vidx, best_ref[1, :])

        @pl.when(c == GRID - 1)
        def _():
            val = best_ref[0, :]
            idx = best_ref[1, :]
            comm_ref[N_DEV - 1, 0, :] = val
            comm_ref[N_DEV - 1, 1, :] = idx

            pl.semaphore_wait(barrier_sem, N_DEV - 1)

            rdmas = []
            for d in range(1, N_DEV):
                peer = lax.rem(my_pos + d, N_DEV)
                rdma = pltpu.make_async_remote_copy(
                    src_ref=comm_ref.at[N_DEV - 1],
                    dst_ref=comm_ref.at[d - 1],
                    send_sem=send_sems.at[d - 1],
                    recv_sem=recv_sems.at[d - 1],
                    device_id=(peer,),
                    device_id_type=pl.DeviceIdType.MESH,
                )
                rdma.start()
                rdmas.append(rdma)

            best_val = val
            best_idx = idx
            for d in (1, 3, 2):
                rdmas[d - 1].wait_recv()
                new_val = comm_ref[d - 1, 0, :]
                new_idx = comm_ref[d - 1, 1, :]
                take = (new_val > best_val) | (
                    (new_val == best_val) & (new_idx < best_idx)
                )
                best_val = jnp.where(take, new_val, best_val)
                best_idx = jnp.where(take, new_idx, best_idx)

            best_ref[0, :] = best_val
            best_ref[1, :] = best_idx
            out_copy = pltpu.make_async_copy(best_ref, out_ref, out_sem)
            out_copy.start()
            out_copy.wait()

            for d in range(1, N_DEV):
                rdmas[d - 1].wait_send()

    return pl.pallas_call(
        body,
        grid=(GRID,),
        out_shape=jax.ShapeDtypeStruct((2, n), jnp.float32),
        in_specs=[pl.BlockSpec((bm, n), lambda c: (c, 0))],
        out_specs=pl.BlockSpec(memory_space=pl.ANY),
        scratch_shapes=[
            pltpu.VMEM((2, n), jnp.float32),
            pltpu.VMEM((N_DEV, 2, n), jnp.float32),
            pltpu.SemaphoreType.DMA((N_DEV - 1,)),
            pltpu.SemaphoreType.DMA((N_DEV - 1,)),
            pltpu.SemaphoreType.DMA,
        ],
        compiler_params=pltpu.CompilerParams(
            collective_id=0,
            dimension_semantics=("arbitrary",),
        ),
    )(x)
